# Initial kernel scaffold; baseline (speedup 1.0000x reference)
import jax
import jax.numpy as jnp
from jax.experimental import pallas as pl


def kernel(x, phrase_mask, phrase_token_idx, phrase_end_pos, phrase_id, norm1_w, Wq1, Wq2, Wk, Wv, Wo, Wqi, Wki, sink_k, sink_v, norm2_w, W1, W2):
    raise NotImplementedError("write your pallas kernel here")


# speedup vs baseline: 5.9242x; 5.9242x over previous
import jax
import jax.numpy as jnp
from jax.experimental import pallas as pl

B, S, D = 1, 2048, 768
H, HD = 8, 64
QC = 256
IH, IHD = 2, 32
TOPK = 8
P, L = 256, 16
WIN = 64
FFM = 4
NEG = -1e9
BS = 256
FFC = 2
FH = (FFM * D) // FFC

_f32 = jnp.float32


def _dot(a, b):
    return jnp.dot(a, b, preferred_element_type=_f32)


def _rms_in(x, w):
    return x * jax.lax.rsqrt(jnp.mean(x * x, axis=-1, keepdims=True) + 1e-6) * w


def _proj_kernel(x_ref, nw_ref, wq1_ref, wq2_ref, wk_ref, wv_ref, wqi_ref,
                 h_ref, q_ref, k_ref, v_ref, qi_ref):
    x = x_ref[...]
    h = _rms_in(x, nw_ref[...])
    h_ref[...] = h
    q_ref[...] = _dot(_dot(h, wq1_ref[...]), wq2_ref[...])
    k_ref[...] = _dot(h, wk_ref[...])
    v_ref[...] = _dot(h, wv_ref[...])
    qi_ref[...] = _dot(h, wqi_ref[...])


def _phrase_kernel(h_ref, idx_ref, wk_ref, wv_ref, wki_ref,
                   pk_ref, pv_ref, ki_ref):
    idx = idx_ref[...]
    iota = jax.lax.broadcasted_iota(jnp.int32, (P, S), 1)
    acc = jnp.zeros((P, S), _f32)
    for l in range(L):
        acc += (idx[:, l:l + 1] == iota).astype(_f32)
    ph = _dot(acc, h_ref[...]) * (1.0 / L)
    pk_ref[...] = _dot(ph, wk_ref[...])
    pv_ref[...] = _dot(ph, wv_ref[...])
    ki_ref[...] = _dot(ph, wki_ref[...])


def _bias_kernel(qi_ref, ki_ref, end_ref, mask_ref, bias_ref):
    i = pl.program_id(0)
    qi = qi_ref[...]
    ki = ki_ref[...]
    scores = jax.lax.dot_general(qi, ki, (((1,), (1,)), ((), ())),
                                 preferred_element_type=_f32)
    pos = i * BS + jax.lax.broadcasted_iota(jnp.int32, (BS, P), 0)
    allowed = (end_ref[...] <= pos) & (mask_ref[...] != 0)
    scores = jnp.where(allowed, scores, NEG)
    r = scores
    thr = jnp.full((BS, 1), NEG, _f32)
    for _ in range(TOPK):
        thr = jnp.max(r, axis=-1, keepdims=True)
        r = jnp.where(r >= thr, NEG, r)
    sel = allowed & (scores >= thr)
    bias_ref[...] = jnp.where(sel, 0.0, NEG)


def _attn_kernel(q_ref, kp_ref, kc_ref, vp_ref, vc_ref, pk_ref, pv_ref,
                 bias_ref, skT_ref, sv_ref, x_ref, wo_ref, o_ref):
    i = pl.program_id(0)
    scale = HD ** -0.5
    q = q_ref[...]
    kp, kc = kp_ref[...], kc_ref[...]
    vp, vc = vp_ref[...], vc_ref[...]
    pk, pv = pk_ref[...], pv_ref[...]
    bias = bias_ref[...]

    srow = jax.lax.broadcasted_iota(jnp.int32, (BS, BS), 0)
    tcol = jax.lax.broadcasted_iota(jnp.int32, (BS, BS), 1)
    dist_p = BS + srow - tcol
    mask_p = (dist_p < WIN) & (i > 0)
    dist_c = srow - tcol
    mask_c = (dist_c >= 0) & (dist_c < WIN)

    ctxs = []
    for h in range(H):
        sl = slice(h * HD, (h + 1) * HD)
        qh = q[:, sl]
        cd = (((1,), (1,)), ((), ()))
        lp = jnp.where(mask_p, jax.lax.dot_general(
            qh, kp[:, sl], cd, preferred_element_type=_f32) * scale, NEG)
        lc = jnp.where(mask_c, jax.lax.dot_general(
            qh, kc[:, sl], cd, preferred_element_type=_f32) * scale, NEG)
        lph = jax.lax.dot_general(
            qh, pk[:, sl], cd, preferred_element_type=_f32) * scale + bias
        ls = _dot(qh, skT_ref[:, h:h + 1]) * scale
        m = jnp.maximum(jnp.maximum(jnp.max(lp, -1, keepdims=True),
                                    jnp.max(lc, -1, keepdims=True)),
                        jnp.maximum(jnp.max(lph, -1, keepdims=True), ls))
        ep = jnp.exp(lp - m)
        ec = jnp.exp(lc - m)
        eph = jnp.exp(lph - m)
        es = jnp.exp(ls - m)
        denom = (jnp.sum(ep, -1, keepdims=True) + jnp.sum(ec, -1, keepdims=True)
                 + jnp.sum(eph, -1, keepdims=True) + es)
        ctx = (_dot(ep, vp[:, sl]) + _dot(ec, vc[:, sl]) + _dot(eph, pv[:, sl])
               + es * sv_ref[h:h + 1, :])
        ctxs.append(ctx / denom)
    ctx_all = jnp.concatenate(ctxs, axis=1)
    o_ref[...] = x_ref[...] + _dot(ctx_all, wo_ref[...])


def _ffn_kernel(x_ref, nw_ref, w1_ref, w2_ref, o_ref):
    j = pl.program_id(1)
    x = x_ref[...]
    h2 = _rms_in(x, nw_ref[...])
    part = _dot(jax.nn.gelu(_dot(h2, w1_ref[...])), w2_ref[...])

    @pl.when(j == 0)
    def _():
        o_ref[...] = x + part

    @pl.when(j != 0)
    def _():
        o_ref[...] += part


def _full(shape):
    n = len(shape)
    return pl.BlockSpec(shape, lambda *a: (0,) * n)


def kernel(x, phrase_mask, phrase_token_idx, phrase_end_pos, phrase_id,
           norm1_w, Wq1, Wq2, Wk, Wv, Wo, Wqi, Wki, sink_k, sink_v,
           norm2_w, W1, W2):
    xs = x.reshape(S, D)
    nw1 = norm1_w.reshape(1, D)
    nw2 = norm2_w.reshape(1, D)
    idx = phrase_token_idx.reshape(P, L).astype(jnp.int32)
    end = phrase_end_pos.reshape(1, P).astype(jnp.int32)
    pmask = phrase_mask.reshape(1, P).astype(jnp.int32)

    nblk = S // BS
    seq = lambda i: (i, 0)

    h, q, k, v, qi = pl.pallas_call(
        _proj_kernel,
        grid=(nblk,),
        in_specs=[pl.BlockSpec((BS, D), seq), _full((1, D)),
                  _full((D, QC)), _full((QC, H * HD)), _full((D, H * HD)),
                  _full((D, H * HD)), _full((D, IH * IHD))],
        out_specs=[pl.BlockSpec((BS, D), seq), pl.BlockSpec((BS, H * HD), seq),
                   pl.BlockSpec((BS, H * HD), seq), pl.BlockSpec((BS, H * HD), seq),
                   pl.BlockSpec((BS, IH * IHD), seq)],
        out_shape=[jax.ShapeDtypeStruct((S, D), _f32),
                   jax.ShapeDtypeStruct((S, H * HD), _f32),
                   jax.ShapeDtypeStruct((S, H * HD), _f32),
                   jax.ShapeDtypeStruct((S, H * HD), _f32),
                   jax.ShapeDtypeStruct((S, IH * IHD), _f32)],
    )(xs, nw1, Wq1, Wq2, Wk, Wv, Wqi)

    pk, pv, ki = pl.pallas_call(
        _phrase_kernel,
        grid=(1,),
        in_specs=[_full((S, D)), _full((P, L)), _full((D, H * HD)),
                  _full((D, H * HD)), _full((D, IH * IHD))],
        out_specs=[_full((P, H * HD)), _full((P, H * HD)), _full((P, IH * IHD))],
        out_shape=[jax.ShapeDtypeStruct((P, H * HD), _f32),
                   jax.ShapeDtypeStruct((P, H * HD), _f32),
                   jax.ShapeDtypeStruct((P, IH * IHD), _f32)],
    )(h, idx, Wk, Wv, Wki)

    bias = pl.pallas_call(
        _bias_kernel,
        grid=(nblk,),
        in_specs=[pl.BlockSpec((BS, IH * IHD), seq), _full((P, IH * IHD)),
                  _full((1, P)), _full((1, P))],
        out_specs=pl.BlockSpec((BS, P), seq),
        out_shape=jax.ShapeDtypeStruct((S, P), _f32),
    )(qi, ki, end, pmask)

    prev = lambda i: (jnp.maximum(i - 1, 0), 0)
    x2 = pl.pallas_call(
        _attn_kernel,
        grid=(nblk,),
        in_specs=[pl.BlockSpec((BS, H * HD), seq),
                  pl.BlockSpec((BS, H * HD), prev), pl.BlockSpec((BS, H * HD), seq),
                  pl.BlockSpec((BS, H * HD), prev), pl.BlockSpec((BS, H * HD), seq),
                  _full((P, H * HD)), _full((P, H * HD)),
                  pl.BlockSpec((BS, P), seq),
                  _full((HD, H)), _full((H, HD)),
                  pl.BlockSpec((BS, D), seq), _full((H * HD, D))],
        out_specs=pl.BlockSpec((BS, D), seq),
        out_shape=jax.ShapeDtypeStruct((S, D), _f32),
    )(q, k, k, v, v, pk, pv, bias, sink_k.T, sink_v, xs, Wo)

    out = pl.pallas_call(
        _ffn_kernel,
        grid=(nblk, FFC),
        in_specs=[pl.BlockSpec((BS, D), lambda i, j: (i, 0)), _full((1, D)),
                  pl.BlockSpec((D, FH), lambda i, j: (0, j)),
                  pl.BlockSpec((FH, D), lambda i, j: (j, 0))],
        out_specs=pl.BlockSpec((BS, D), lambda i, j: (i, 0)),
        out_shape=jax.ShapeDtypeStruct((S, D), _f32),
    )(x2, nw2, W1, W2)

    return out.reshape(B, S, D)


# speedup vs baseline: 5.9808x; 1.0096x over previous
import jax
import jax.numpy as jnp
from jax.experimental import pallas as pl

B, S, D = 1, 2048, 768
H, HD = 8, 64
QC = 256
IH, IHD = 2, 32
TOPK = 8
P, L = 256, 16
WIN = 64
FFM = 4
NEG = -1e9
BS = 256
FFC = 2
FH = (FFM * D) // FFC

_f32 = jnp.float32


def _dot(a, b):
    return jnp.dot(a, b, preferred_element_type=_f32)


def _bdot(a, b):
    return jnp.dot(a.astype(jnp.bfloat16), b.astype(jnp.bfloat16),
                   preferred_element_type=_f32)


def _bdot_nt(a, b):
    return jax.lax.dot_general(a.astype(jnp.bfloat16), b.astype(jnp.bfloat16),
                               (((1,), (1,)), ((), ())),
                               preferred_element_type=_f32)


def _rms_in(x, w):
    return x * jax.lax.rsqrt(jnp.mean(x * x, axis=-1, keepdims=True) + 1e-6) * w


def _proj_kernel(x_ref, nw_ref, wq1_ref, wq2_ref, wk_ref, wv_ref, wqi_ref,
                 h_ref, q_ref, k_ref, v_ref, qi_ref):
    x = x_ref[...]
    h = _rms_in(x, nw_ref[...])
    h_ref[...] = h
    q_ref[...] = _bdot(_bdot(h, wq1_ref[...]), wq2_ref[...])
    k_ref[...] = _bdot(h, wk_ref[...])
    v_ref[...] = _bdot(h, wv_ref[...])
    qi_ref[...] = _dot(h, wqi_ref[...])


def _phrase_kernel(h_ref, idx_ref, wk_ref, wv_ref, wki_ref,
                   pk_ref, pv_ref, ki_ref):
    idx = idx_ref[...]
    iota = jax.lax.broadcasted_iota(jnp.int32, (P, S), 1)
    acc = jnp.zeros((P, S), _f32)
    for l in range(L):
        acc += (idx[:, l:l + 1] == iota).astype(_f32)
    ph = _dot(acc, h_ref[...]) * (1.0 / L)
    pk_ref[...] = _bdot(ph, wk_ref[...])
    pv_ref[...] = _bdot(ph, wv_ref[...])
    ki_ref[...] = _dot(ph, wki_ref[...])


def _bias_kernel(qi_ref, ki_ref, end_ref, mask_ref, bias_ref):
    i = pl.program_id(0)
    qi = qi_ref[...]
    ki = ki_ref[...]
    scores = jax.lax.dot_general(qi, ki, (((1,), (1,)), ((), ())),
                                 preferred_element_type=_f32)
    pos = i * BS + jax.lax.broadcasted_iota(jnp.int32, (BS, P), 0)
    allowed = (end_ref[...] <= pos) & (mask_ref[...] != 0)
    scores = jnp.where(allowed, scores, NEG)
    r = scores
    thr = jnp.full((BS, 1), NEG, _f32)
    for _ in range(TOPK):
        thr = jnp.max(r, axis=-1, keepdims=True)
        r = jnp.where(r >= thr, NEG, r)
    sel = allowed & (scores >= thr)
    bias_ref[...] = jnp.where(sel, 0.0, NEG)


def _attn_kernel(q_ref, kp_ref, kc_ref, vp_ref, vc_ref, pk_ref, pv_ref,
                 bias_ref, skT_ref, sv_ref, x_ref, wo_ref, o_ref):
    i = pl.program_id(0)
    scale = HD ** -0.5
    q = q_ref[...]
    kp, kc = kp_ref[...], kc_ref[...]
    vp, vc = vp_ref[...], vc_ref[...]
    pk, pv = pk_ref[...], pv_ref[...]
    bias = bias_ref[...]

    srow = jax.lax.broadcasted_iota(jnp.int32, (BS, BS), 0)
    tcol = jax.lax.broadcasted_iota(jnp.int32, (BS, BS), 1)
    dist_p = BS + srow - tcol
    mask_p = (dist_p < WIN) & (i > 0)
    dist_c = srow - tcol
    mask_c = (dist_c >= 0) & (dist_c < WIN)

    ctxs = []
    for h in range(H):
        sl = slice(h * HD, (h + 1) * HD)
        qh = q[:, sl]
        lp = jnp.where(mask_p, _bdot_nt(qh, kp[:, sl]) * scale, NEG)
        lc = jnp.where(mask_c, _bdot_nt(qh, kc[:, sl]) * scale, NEG)
        lph = _bdot_nt(qh, pk[:, sl]) * scale + bias
        ls = _bdot(qh, skT_ref[:, h:h + 1]) * scale
        m = jnp.maximum(jnp.maximum(jnp.max(lp, -1, keepdims=True),
                                    jnp.max(lc, -1, keepdims=True)),
                        jnp.maximum(jnp.max(lph, -1, keepdims=True), ls))
        ep = jnp.exp(lp - m)
        ec = jnp.exp(lc - m)
        eph = jnp.exp(lph - m)
        es = jnp.exp(ls - m)
        denom = (jnp.sum(ep, -1, keepdims=True) + jnp.sum(ec, -1, keepdims=True)
                 + jnp.sum(eph, -1, keepdims=True) + es)
        ctx = (_bdot(ep, vp[:, sl]) + _bdot(ec, vc[:, sl]) + _bdot(eph, pv[:, sl])
               + es * sv_ref[h:h + 1, :])
        ctxs.append(ctx / denom)
    ctx_all = jnp.concatenate(ctxs, axis=1)
    o_ref[...] = x_ref[...] + _bdot(ctx_all, wo_ref[...])


def _ffn_kernel(x_ref, nw_ref, w1_ref, w2_ref, o_ref):
    j = pl.program_id(1)
    x = x_ref[...]
    h2 = _rms_in(x, nw_ref[...])
    part = _bdot(jax.nn.gelu(_bdot(h2, w1_ref[...])), w2_ref[...])

    @pl.when(j == 0)
    def _():
        o_ref[...] = x + part

    @pl.when(j != 0)
    def _():
        o_ref[...] += part


def _full(shape):
    n = len(shape)
    return pl.BlockSpec(shape, lambda *a: (0,) * n)


def kernel(x, phrase_mask, phrase_token_idx, phrase_end_pos, phrase_id,
           norm1_w, Wq1, Wq2, Wk, Wv, Wo, Wqi, Wki, sink_k, sink_v,
           norm2_w, W1, W2):
    xs = x.reshape(S, D)
    nw1 = norm1_w.reshape(1, D)
    nw2 = norm2_w.reshape(1, D)
    idx = phrase_token_idx.reshape(P, L).astype(jnp.int32)
    end = phrase_end_pos.reshape(1, P).astype(jnp.int32)
    pmask = phrase_mask.reshape(1, P).astype(jnp.int32)

    nblk = S // BS
    seq = lambda i: (i, 0)

    h, q, k, v, qi = pl.pallas_call(
        _proj_kernel,
        grid=(nblk,),
        in_specs=[pl.BlockSpec((BS, D), seq), _full((1, D)),
                  _full((D, QC)), _full((QC, H * HD)), _full((D, H * HD)),
                  _full((D, H * HD)), _full((D, IH * IHD))],
        out_specs=[pl.BlockSpec((BS, D), seq), pl.BlockSpec((BS, H * HD), seq),
                   pl.BlockSpec((BS, H * HD), seq), pl.BlockSpec((BS, H * HD), seq),
                   pl.BlockSpec((BS, IH * IHD), seq)],
        out_shape=[jax.ShapeDtypeStruct((S, D), _f32),
                   jax.ShapeDtypeStruct((S, H * HD), _f32),
                   jax.ShapeDtypeStruct((S, H * HD), _f32),
                   jax.ShapeDtypeStruct((S, H * HD), _f32),
                   jax.ShapeDtypeStruct((S, IH * IHD), _f32)],
    )(xs, nw1, Wq1, Wq2, Wk, Wv, Wqi)

    pk, pv, ki = pl.pallas_call(
        _phrase_kernel,
        grid=(1,),
        in_specs=[_full((S, D)), _full((P, L)), _full((D, H * HD)),
                  _full((D, H * HD)), _full((D, IH * IHD))],
        out_specs=[_full((P, H * HD)), _full((P, H * HD)), _full((P, IH * IHD))],
        out_shape=[jax.ShapeDtypeStruct((P, H * HD), _f32),
                   jax.ShapeDtypeStruct((P, H * HD), _f32),
                   jax.ShapeDtypeStruct((P, IH * IHD), _f32)],
    )(h, idx, Wk, Wv, Wki)

    bias = pl.pallas_call(
        _bias_kernel,
        grid=(nblk,),
        in_specs=[pl.BlockSpec((BS, IH * IHD), seq), _full((P, IH * IHD)),
                  _full((1, P)), _full((1, P))],
        out_specs=pl.BlockSpec((BS, P), seq),
        out_shape=jax.ShapeDtypeStruct((S, P), _f32),
    )(qi, ki, end, pmask)

    prev = lambda i: (jnp.maximum(i - 1, 0), 0)
    x2 = pl.pallas_call(
        _attn_kernel,
        grid=(nblk,),
        in_specs=[pl.BlockSpec((BS, H * HD), seq),
                  pl.BlockSpec((BS, H * HD), prev), pl.BlockSpec((BS, H * HD), seq),
                  pl.BlockSpec((BS, H * HD), prev), pl.BlockSpec((BS, H * HD), seq),
                  _full((P, H * HD)), _full((P, H * HD)),
                  pl.BlockSpec((BS, P), seq),
                  _full((HD, H)), _full((H, HD)),
                  pl.BlockSpec((BS, D), seq), _full((H * HD, D))],
        out_specs=pl.BlockSpec((BS, D), seq),
        out_shape=jax.ShapeDtypeStruct((S, D), _f32),
    )(q, k, k, v, v, pk, pv, bias, sink_k.T, sink_v, xs, Wo)

    out = pl.pallas_call(
        _ffn_kernel,
        grid=(nblk, FFC),
        in_specs=[pl.BlockSpec((BS, D), lambda i, j: (i, 0)), _full((1, D)),
                  pl.BlockSpec((D, FH), lambda i, j: (0, j)),
                  pl.BlockSpec((FH, D), lambda i, j: (j, 0))],
        out_specs=pl.BlockSpec((BS, D), lambda i, j: (i, 0)),
        out_shape=jax.ShapeDtypeStruct((S, D), _f32),
    )(x2, nw2, W1, W2)

    return out.reshape(B, S, D)
